# baseline (speedup 1.0000x reference)
import functools

import jax
import jax.numpy as jnp
from jax import lax
from jax.experimental import pallas as pl
from jax.experimental.pallas import tpu as pltpu
from jax.experimental.pallas import tpu_sc as plsc

N_NODES = 32000
N_EDGES = 1024000
N_GRAPHS = 32
NF, EF, UF, HID = 16, 8, 32, 64
OUT_COLS = 499500

_NC, _NS = 2, 16
_NW = _NC * _NS
_IW = 128
_EPW = N_EDGES // _NW
_RPW = _EPW // _IW
_NB = 10
_NG = _RPW // _NB
_EROWS = N_EDGES // _IW

_f32 = jnp.float32
_i32 = jnp.int32


def _sc_mesh():
    return plsc.VectorSubcoreMesh(core_axis_name="c", subcore_axis_name="s")


def _gather_rows_loop(tbl, idxv, out_hbm, rows, sem, r0):
    def grp(g, carry):
        cps = []
        for b in range(_NB):
            cps.append(pltpu.async_copy(tbl.at[idxv.at[g * _NB + b]],
                                        rows.at[b], sem))
        for cp in cps:
            cp.wait()
        pltpu.sync_copy(rows, out_hbm.at[pl.ds(r0 + g * _NB, _NB)])
        return carry
    lax.fori_loop(0, _NG, grp, 0)


def _make_gather():
    out_type = jax.ShapeDtypeStruct((2, _EROWS, _IW, NF), _f32)
    scratch = [pltpu.VMEM((_RPW, _IW), _i32),
               pltpu.VMEM((_NB, _IW, NF), _f32),
               pltpu.VMEM_SHARED((N_NODES, NF), _f32),
               pltpu.SemaphoreType.DMA]

    def body(x_r, src_r, dst_r, xg_o, idxv, rows, shx, sem):
        s = lax.axis_index("s")
        w = s * _NC + lax.axis_index("c")
        r0 = w * _RPW

        @pl.when(s == 0)
        def _():
            pltpu.sync_copy(x_r, shx)
        plsc.subcore_barrier()
        pltpu.sync_copy(src_r.at[w], idxv)
        _gather_rows_loop(shx, idxv, xg_o.at[0], rows, sem, r0)
        pltpu.sync_copy(dst_r.at[w], idxv)
        _gather_rows_loop(shx, idxv, xg_o.at[1], rows, sem, r0)

    return pl.kernel(body, out_type=out_type, mesh=_sc_mesh(),
                     scratch_types=scratch,
                     compiler_params=pltpu.CompilerParams(
                         use_tc_tiling_on_sc=False))


def _make_scatter():
    out_type = jax.ShapeDtypeStruct((_NC, N_NODES, 16), _f32)
    scratch = [pltpu.VMEM((_RPW, _IW), _i32),
               pltpu.VMEM((_NB, _IW, 16), _f32),
               pltpu.VMEM_SHARED((N_NODES, 16), _f32)]

    def body(vals_r, dst_r, zeros_r, out_r, idxv, valsb, shared):
        c = lax.axis_index("c")
        s = lax.axis_index("s")
        w = s * _NC + c
        r0 = w * _RPW

        @pl.when(s == 0)
        def _():
            pltpu.sync_copy(zeros_r, shared)
        plsc.subcore_barrier()

        pltpu.sync_copy(dst_r.at[w], idxv)

        def grp(g, carry):
            pltpu.sync_copy(vals_r.at[pl.ds(r0 + g * _NB, _NB)], valsb)
            for b in range(_NB):
                pltpu.sync_copy(valsb.at[b], shared.at[idxv.at[g * _NB + b]],
                                add=True)
            return carry
        lax.fori_loop(0, _NG, grp, 0)

        plsc.subcore_barrier()

        @pl.when(s == 0)
        def _():
            pltpu.sync_copy(shared, out_r.at[c])

    return pl.kernel(body, out_type=out_type, mesh=_sc_mesh(),
                     scratch_types=scratch,
                     compiler_params=pltpu.CompilerParams(
                         use_tc_tiling_on_sc=False))


_BE = 4096
_BN = 3200
_BC = 8192


def _dot(a, b):
    return jnp.dot(a, b, preferred_element_type=_f32)


_PK = 8
_PROWS = N_EDGES // _PK
_BPR = _BE // _PK


def _full(shape):
    nd = len(shape)
    return pl.BlockSpec(shape, lambda i, _n=nd: (0,) * _n)


_bf16 = jnp.bfloat16


def _bdot(a, b):
    return jnp.dot(a.astype(_bf16), b.astype(_bf16),
                   preferred_element_type=_f32)


def _edge1_body(xg, eap, srcp, wu, lo_r, hi_r, rep,
                wxd, we, b1, w2, b2, o):
    v = xg[...]
    xsd = jnp.concatenate([v[0], v[1]], axis=1)
    srcb = _dot(srcp[...], rep[...])
    oh = ((srcb >= lo_r[...]) & (srcb < hi_r[...])).astype(_f32)
    h = (_bdot(xsd, wxd[...])
         + _bdot(eap[...], we[...]) + _bdot(oh, wu[...]) + b1[...])
    h = jnp.maximum(h, 0.0)
    o[...] = _bdot(h, w2[...]) + b2[...]


def _edge23_body(xg, v1, srcp, wu, lo_r, hi_r, rep,
                 wxd, we, b1, w2, b2, o):
    v = xg[...]
    xsd = jnp.concatenate([v[0], v[1]], axis=1)
    srcb = _dot(srcp[...], rep[...])
    oh = ((srcb >= lo_r[...]) & (srcb < hi_r[...])).astype(_f32)
    e1 = jnp.maximum(v1[...], 0.0)
    h = (_bdot(xsd, wxd[...])
         + _bdot(e1, we[...]) + _bdot(oh, wu[...]) + b1[...])
    h = jnp.maximum(h, 0.0)
    o[...] = _bdot(h, w2[...]) + b2[...]


def _edge_call(body, extra, extra_spec, weights, xgp, srcp, wu, lo_r, hi_r,
               rep):
    wspecs = [_full(w.shape) for w in weights]
    return pl.pallas_call(
        body,
        grid=(N_EDGES // _BE,),
        in_specs=([pl.BlockSpec((2, _BPR, 128), lambda i: (0, i, 0)),
                   extra_spec,
                   pl.BlockSpec((_BPR, _PK), lambda i: (i, 0)),
                   _full(wu.shape), _full((1, 32 * _PK)),
                   _full((1, 32 * _PK)), _full((_PK, 32 * _PK))] + wspecs),
        out_specs=pl.BlockSpec((_BPR, 128), lambda i: (i, 0)),
        out_shape=jax.ShapeDtypeStruct((_PROWS, 128), _f32),
    )(xgp, extra, srcp, wu, lo_r, hi_r, rep, *weights)


def _uprep_body(u, w1u, Vc, Hc, Mc, lo, hi, rg, wu_o, lo_o, hi_o):
    uproj = _dot(u[...], w1u[...])
    wu_o[...] = _dot(_dot(Vc[...], uproj), Hc[...]) * Mc[...]
    lo_o[...] = _dot(lo[...].astype(_f32), rg[...])
    hi_o[...] = _dot(hi[...].astype(_f32), rg[...])


def _uprep_call(u_in, w1u, Vc, Hc, Mc, lo, hi, rg):
    hid = w1u.shape[1]
    return pl.pallas_call(
        _uprep_body,
        grid=(1,),
        in_specs=[_full(u_in.shape), _full(w1u.shape), _full(Vc.shape),
                  _full(Hc.shape), _full(Mc.shape), _full((1, N_GRAPHS)),
                  _full((1, N_GRAPHS)), _full(rg.shape)],
        out_specs=[_full((32 * _PK, _PK * hid)), _full((1, 32 * _PK)),
                   _full((1, 32 * _PK))],
        out_shape=[jax.ShapeDtypeStruct((32 * _PK, _PK * hid), _f32),
                   jax.ShapeDtypeStruct((1, 32 * _PK), _f32),
                   jax.ShapeDtypeStruct((1, 32 * _PK), _f32)],
    )(u_in, w1u, Vc, Hc, Mc, lo, hi, rg)


_BB = 3200


def _bounds_body(bcol, tri, lo_o, hi_o, C):
    i = pl.program_id(0)
    oh = (bcol[...] == lax.broadcasted_iota(_i32, (_BB, N_GRAPHS), 1)
          ).astype(_f32)

    @pl.when(i == 0)
    def _():
        C[...] = jnp.zeros_like(C)

    C[...] += jnp.sum(oh, axis=0, keepdims=True)

    @pl.when(i == pl.num_programs(0) - 1)
    def _():
        cnt = C[...]
        lo = _dot(cnt, tri[...])
        lo_o[...] = lo.astype(_i32)
        hi_o[...] = (lo + cnt).astype(_i32)


def _bounds_call(bcol):
    tri = (lax.broadcasted_iota(_i32, (N_GRAPHS, N_GRAPHS), 0)
           < lax.broadcasted_iota(_i32, (N_GRAPHS, N_GRAPHS), 1)).astype(_f32)
    return pl.pallas_call(
        _bounds_body,
        grid=(N_NODES // _BB,),
        in_specs=[pl.BlockSpec((_BB, 1), lambda i: (i, 0)),
                  _full((N_GRAPHS, N_GRAPHS))],
        out_specs=[_full((1, N_GRAPHS)), _full((1, N_GRAPHS))],
        out_shape=[jax.ShapeDtypeStruct((1, N_GRAPHS), _i32),
                   jax.ShapeDtypeStruct((1, N_GRAPHS), _i32)],
        scratch_shapes=[pltpu.VMEM((1, N_GRAPHS), _f32)],
    )(bcol, tri)


def _node1_body(xr, aggr, bcol, bt8, u,
                w1x, w1a, w1u, b1, w2, b2, wgu, wgx, bg1, wg2, bg2,
                x1_o, u1_o, S, C):
    i = pl.program_id(0)
    a2 = aggr[...]
    aggs = a2[0] + a2[1]
    deg = jnp.maximum(aggs[:, 8:9], 1.0)
    aggn = aggs / deg
    oh = (bcol[...] == lax.broadcasted_iota(_i32, (_BN, N_GRAPHS), 1)
          ).astype(_f32)
    uproj = _dot(u[...], w1u[...])
    h = (_dot(xr[...], w1x[...]) + _dot(aggn, w1a[...]) + _dot(oh, uproj)
         + b1[...])
    h = jnp.maximum(h, 0.0)
    x2 = _dot(h, w2[...]) + b2[...]
    x1_o[...] = jnp.maximum(x2, 0.0)

    ohT = (lax.broadcasted_iota(_i32, (N_GRAPHS, _BN), 0) == bt8[0:1, :]
           ).astype(_f32)

    @pl.when(i == 0)
    def _():
        S[...] = jnp.zeros_like(S)
        C[...] = jnp.zeros_like(C)

    S[...] += _dot(ohT, x2)
    C[...] += jnp.sum(ohT, axis=1, keepdims=True)

    @pl.when(i == pl.num_programs(0) - 1)
    def _():
        Sm = S[...] / jnp.maximum(C[...], 1.0)
        hg = jnp.maximum(_dot(u[...], wgu[...]) + _dot(Sm, wgx[...])
                         + bg1[...], 0.0)
        u1_o[...] = jnp.maximum(_dot(hg, wg2[...]) + bg2[...], 0.0)


def _node23_body(xr, agg1r, agg2r, bcol, bt8, u,
                 nx2, na2, nu2, nb2, nw2, nc2,
                 nx3, na3, nu3, nb3, nw3, nc3,
                 gu2, gx2, gb2, gw2, gc2,
                 gu3, gx3, gb3, gw3, gc3,
                 mu_o, lv_o, S2, S3, C):
    i = pl.program_id(0)
    a1 = agg1r[...]
    deg = jnp.maximum((a1[0] + a1[1])[:, 8:9], 1.0)
    a2 = agg2r[...]
    a = (a2[0] + a2[1]) / deg
    oh = (bcol[...] == lax.broadcasted_iota(_i32, (_BN, N_GRAPHS), 1)
          ).astype(_f32)
    ohT = (lax.broadcasted_iota(_i32, (N_GRAPHS, _BN), 0) == bt8[0:1, :]
           ).astype(_f32)
    xv = xr[...]
    uproj2 = _dot(u[...], nu2[...])
    h2 = jnp.maximum(_dot(xv, nx2[...]) + _dot(a, na2[...]) + _dot(oh, uproj2)
                     + nb2[...], 0.0)
    x2b = _dot(h2, nw2[...]) + nc2[...]
    uproj3 = _dot(u[...], nu3[...])
    h3 = jnp.maximum(_dot(xv, nx3[...]) + _dot(a, na3[...]) + _dot(oh, uproj3)
                     + nb3[...], 0.0)
    x3b = _dot(h3, nw3[...]) + nc3[...]

    @pl.when(i == 0)
    def _():
        S2[...] = jnp.zeros_like(S2)
        S3[...] = jnp.zeros_like(S3)
        C[...] = jnp.zeros_like(C)

    S2[...] += _dot(ohT, x2b)
    S3[...] += _dot(ohT, x3b)
    C[...] += jnp.sum(ohT, axis=1, keepdims=True)

    @pl.when(i == pl.num_programs(0) - 1)
    def _():
        cc = jnp.maximum(C[...], 1.0)
        Sm2 = S2[...] / cc
        hg2 = jnp.maximum(_dot(u[...], gu2[...]) + _dot(Sm2, gx2[...])
                          + gb2[...], 0.0)
        mu_o[...] = _dot(hg2, gw2[...]) + gc2[...]
        Sm3 = S3[...] / cc
        hg3 = jnp.maximum(_dot(u[...], gu3[...]) + _dot(Sm3, gx3[...])
                          + gb3[...], 0.0)
        lv_o[...] = _dot(hg3, gw3[...]) + gc3[...]


def _dec_body(mu, lv, ep, w1, b1, w2, b2, o):
    std = jnp.exp(0.5 * lv[...])
    z = mu[...] + ep[...] * std
    h = jnp.maximum(_dot(z, w1[...]) + b1[...], 0.0)
    o[...] = jax.nn.sigmoid(_dot(h, w2[...]) + b2[...])


def _split_edge_w(W1):
    return W1[0:NF], W1[NF:2 * NF], W1[2 * NF:2 * NF + EF], W1[2 * NF + EF:]


def kernel(x, edge_index, edge_attr, u, batch, eps, params):
    src = edge_index[0].reshape(_NW, _RPW, _IW)
    dst = edge_index[1].reshape(_NW, _RPW, _IW)
    zeros16 = jnp.zeros((N_NODES, 16), _f32)
    bcol = batch.reshape(N_NODES, 1)
    bt8 = jnp.broadcast_to(batch.reshape(1, N_NODES), (8, N_NODES))

    eyeP = jnp.eye(_PK, dtype=_f32)

    def bd(w):
        return jnp.kron(eyeP, w)

    def tile_row(b):
        return jnp.tile(b.reshape(1, -1), (1, _PK))

    (e1W1, e1b1), (e1W2, e1b2) = params['gc1']['edge']
    w1x1, w1d1, w1e1, w1u1 = _split_edge_w(e1W1)
    w2p1 = jnp.concatenate([e1W2, jnp.zeros((HID, 8), _f32)], axis=1)
    b2p1 = jnp.concatenate([e1b2, jnp.ones((1,), _f32),
                            jnp.zeros((7,), _f32)])
    ew1 = [jnp.concatenate([bd(w1x1), bd(w1d1)], axis=0), bd(w1e1),
           tile_row(e1b1), bd(w2p1), tile_row(b2p1)]

    (e2W1, e2b1), (e2W2, e2b2) = params['gc2']['edge']
    (e3W1, e3b1), (e3W2, e3b2) = params['gc3']['edge']
    x2s, d2s, ee2, uu2 = _split_edge_w(e2W1)
    x3s, d3s, ee3, uu3 = _split_edge_w(e3W1)
    z64 = jnp.zeros((HID, 8), _f32)
    w1x23 = jnp.concatenate([x2s, x3s], axis=1)
    w1d23 = jnp.concatenate([d2s, d3s], axis=1)
    w1e23 = jnp.concatenate([jnp.concatenate([ee2, ee3], axis=1),
                             jnp.zeros((8, 2 * HID), _f32)], axis=0)
    w1u23 = jnp.concatenate([uu2, uu3], axis=1)
    b123 = jnp.concatenate([e2b1, e3b1])
    w2bd23 = jnp.concatenate([jnp.concatenate([e2W2, z64], axis=1),
                              jnp.concatenate([z64, e3W2], axis=1)], axis=0)
    b223 = jnp.concatenate([e2b2, e3b2])
    ew23 = [jnp.concatenate([bd(w1x23), bd(w1d23)], axis=0), bd(w1e23),
            tile_row(b123), bd(w2bd23), tile_row(b223)]

    Vc = jnp.kron(jnp.eye(N_GRAPHS, dtype=_f32), jnp.ones((_PK, 1), _f32))
    H1c = jnp.kron(jnp.ones((1, _PK), _f32), jnp.eye(HID, dtype=_f32))
    M1c = jnp.kron(jnp.ones((N_GRAPHS, 1), _f32),
                   jnp.kron(eyeP, jnp.ones((1, HID), _f32)))
    H23c = jnp.kron(jnp.ones((1, _PK), _f32),
                    jnp.eye(2 * HID, dtype=_f32))
    M23c = jnp.kron(jnp.ones((N_GRAPHS, 1), _f32),
                    jnp.kron(eyeP, jnp.ones((1, 2 * HID), _f32)))
    RGc = jnp.kron(jnp.eye(N_GRAPHS, dtype=_f32), jnp.ones((1, _PK), _f32))
    REPc = jnp.kron(jnp.ones((1, N_GRAPHS), _f32), eyeP)

    def node_w(p):
        (W1, b1), (W2, b2) = p
        return (W1[0:NF], W1[NF:NF + EF], W1[NF + EF:],
                b1.reshape(1, HID), W2, b2.reshape(1, NF))

    def glob_w(p):
        (W1, b1), (W2, b2) = p
        return (W1[0:UF], W1[UF:], b1.reshape(1, HID), W2,
                b2.reshape(1, UF))

    z8 = jnp.zeros((8, HID), _f32)
    nx1, na1, nu1, nb1, nw1, nc1 = node_w(params['gc1']['node'])
    na1p = jnp.concatenate([na1, z8], axis=0)
    g1 = glob_w(params['gc1']['global'])

    nx2, na2, nu2, nb2, nw2, nc2 = node_w(params['gc2']['node'])
    na2p = jnp.concatenate([na2, z8], axis=0)
    nx3, na3, nu3, nb3, nw3, nc3 = node_w(params['gc3']['node'])
    na3p = jnp.concatenate([z8, na3], axis=0)
    g2 = glob_w(params['gc2']['global'])
    g3 = glob_w(params['gc3']['global'])

    fc1W, fc1b = params['fc1']
    fc2W, fc2b = params['fc2']

    lo, hi = _bounds_call(bcol)
    wu1, lo_r, hi_r = _uprep_call(u, w1u1, Vc, H1c, M1c, lo, hi, RGc)
    srcp = edge_index[0].astype(_f32).reshape(_PROWS, _PK)
    eap = edge_attr.reshape(_PROWS, EF * _PK)
    gather = _make_gather()
    scatter = _make_scatter()
    xg1 = gather(x, src, dst).reshape(2, _PROWS, 128)
    ea_spec = pl.BlockSpec((_BPR, EF * _PK), lambda i: (i, 0))
    vals1 = _edge_call(_edge1_body, eap, ea_spec, ew1, xg1, srcp,
                       wu1, lo_r, hi_r, REPc)
    agg1 = scatter(vals1.reshape(_EROWS, _IW, 16), dst, zeros16)

    nspecs = [pl.BlockSpec((_BN, NF), lambda i: (i, 0)),
              pl.BlockSpec((2, _BN, 16), lambda i: (0, i, 0)),
              pl.BlockSpec((_BN, 1), lambda i: (i, 0)),
              pl.BlockSpec((8, _BN), lambda i: (0, i))]
    w_b1 = [nx1, na1p, nu1, nb1, nw1, nc1] + list(g1)
    x1, u1 = pl.pallas_call(
        _node1_body,
        grid=(N_NODES // _BN,),
        in_specs=nspecs + [_full(u.shape)] + [_full(w.shape) for w in w_b1],
        out_specs=[pl.BlockSpec((_BN, NF), lambda i: (i, 0)),
                   _full((N_GRAPHS, UF))],
        out_shape=[jax.ShapeDtypeStruct((N_NODES, NF), _f32),
                   jax.ShapeDtypeStruct((N_GRAPHS, UF), _f32)],
        scratch_shapes=[pltpu.VMEM((N_GRAPHS, NF), _f32),
                        pltpu.VMEM((N_GRAPHS, 1), _f32)],
    )(x, agg1, bcol, bt8, u, *w_b1)

    wu23, _, _ = _uprep_call(u1, w1u23, Vc, H23c, M23c, lo, hi, RGc)
    xg2 = gather(x1, src, dst).reshape(2, _PROWS, 128)
    v1_spec = pl.BlockSpec((_BPR, 128), lambda i: (i, 0))
    vals2 = _edge_call(_edge23_body, vals1, v1_spec, ew23, xg2, srcp,
                       wu23, lo_r, hi_r, REPc)
    agg2 = scatter(vals2.reshape(_EROWS, _IW, 16), dst, zeros16)

    w_b2 = [nx2, na2p, nu2, nb2, nw2, nc2,
            nx3, na3p, nu3, nb3, nw3, nc3] + list(g2) + list(g3)
    n2specs = [pl.BlockSpec((_BN, NF), lambda i: (i, 0)),
               pl.BlockSpec((2, _BN, 16), lambda i: (0, i, 0)),
               pl.BlockSpec((2, _BN, 16), lambda i: (0, i, 0)),
               pl.BlockSpec((_BN, 1), lambda i: (i, 0)),
               pl.BlockSpec((8, _BN), lambda i: (0, i))]
    mu, logvar = pl.pallas_call(
        _node23_body,
        grid=(N_NODES // _BN,),
        in_specs=n2specs + [_full(u.shape)] + [_full(w.shape) for w in w_b2],
        out_specs=[_full((N_GRAPHS, UF)), _full((N_GRAPHS, UF))],
        out_shape=[jax.ShapeDtypeStruct((N_GRAPHS, UF), _f32),
                   jax.ShapeDtypeStruct((N_GRAPHS, UF), _f32)],
        scratch_shapes=[pltpu.VMEM((N_GRAPHS, NF), _f32),
                        pltpu.VMEM((N_GRAPHS, NF), _f32),
                        pltpu.VMEM((N_GRAPHS, 1), _f32)],
    )(x1, agg1, agg2, bcol, bt8, u1, *w_b2)

    out = pl.pallas_call(
        _dec_body,
        grid=(pl.cdiv(OUT_COLS, _BC),),
        in_specs=[_full((N_GRAPHS, UF)), _full((N_GRAPHS, UF)),
                  _full((N_GRAPHS, UF)), _full(fc1W.shape),
                  _full((1, 100)),
                  pl.BlockSpec((100, _BC), lambda i: (0, i)),
                  pl.BlockSpec((1, _BC), lambda i: (0, i))],
        out_specs=pl.BlockSpec((N_GRAPHS, _BC), lambda i: (0, i)),
        out_shape=jax.ShapeDtypeStruct((N_GRAPHS, OUT_COLS), _f32),
    )(mu, logvar, eps, fc1W, fc1b.reshape(1, 100), fc2W,
      fc2b.reshape(1, OUT_COLS))

    return out, mu, logvar


# speedup vs baseline: 1.0827x; 1.0827x over previous
import functools

import jax
import jax.numpy as jnp
from jax import lax
from jax.experimental import pallas as pl
from jax.experimental.pallas import tpu as pltpu
from jax.experimental.pallas import tpu_sc as plsc

N_NODES = 32000
N_EDGES = 1024000
N_GRAPHS = 32
NF, EF, UF, HID = 16, 8, 32, 64
OUT_COLS = 499500

_NC, _NS = 2, 16
_NW = _NC * _NS
_IW = 128
_EPW = N_EDGES // _NW
_RPW = _EPW // _IW
_NB = 10
_NG = _RPW // _NB
_EROWS = N_EDGES // _IW

_f32 = jnp.float32
_i32 = jnp.int32


def _sc_mesh():
    return plsc.VectorSubcoreMesh(core_axis_name="c", subcore_axis_name="s")


def _gather_rows_loop(tbl, idxv, out_hbm, rows, sem, r0):
    def grp(g, carry):
        cps = []
        for b in range(_NB):
            cps.append(pltpu.async_copy(tbl.at[idxv.at[g * _NB + b]],
                                        rows.at[b], sem))
        for cp in cps:
            cp.wait()
        pltpu.sync_copy(rows, out_hbm.at[pl.ds(r0 + g * _NB, _NB)])
        return carry
    lax.fori_loop(0, _NG, grp, 0)


def _make_gather():
    out_type = jax.ShapeDtypeStruct((2, _EROWS, _IW, NF), _f32)
    scratch = [pltpu.VMEM((_RPW, _IW), _i32),
               pltpu.VMEM((_NB, _IW, NF), _f32),
               pltpu.VMEM_SHARED((N_NODES, NF), _f32),
               pltpu.SemaphoreType.DMA]

    def body(x_r, src_r, dst_r, xg_o, idxv, rows, shx, sem):
        s = lax.axis_index("s")
        w = s * _NC + lax.axis_index("c")
        r0 = w * _RPW

        @pl.when(s == 0)
        def _():
            pltpu.sync_copy(x_r, shx)
        plsc.subcore_barrier()
        pltpu.sync_copy(src_r.at[w], idxv)
        _gather_rows_loop(shx, idxv, xg_o.at[0], rows, sem, r0)
        pltpu.sync_copy(dst_r.at[w], idxv)
        _gather_rows_loop(shx, idxv, xg_o.at[1], rows, sem, r0)

    return pl.kernel(body, out_type=out_type, mesh=_sc_mesh(),
                     scratch_types=scratch,
                     compiler_params=pltpu.CompilerParams(
                         use_tc_tiling_on_sc=False))


def _make_scatter():
    out_type = jax.ShapeDtypeStruct((_NC, N_NODES, 16), _f32)
    scratch = [pltpu.VMEM((_RPW, _IW), _i32),
               pltpu.VMEM((_NB, _IW, 16), _f32),
               pltpu.VMEM_SHARED((N_NODES, 16), _f32)]

    def body(vals_r, dst_r, zeros_r, out_r, idxv, valsb, shared):
        c = lax.axis_index("c")
        s = lax.axis_index("s")
        w = s * _NC + c
        r0 = w * _RPW

        @pl.when(s == 0)
        def _():
            pltpu.sync_copy(zeros_r, shared)
        plsc.subcore_barrier()

        pltpu.sync_copy(dst_r.at[w], idxv)

        def grp(g, carry):
            pltpu.sync_copy(vals_r.at[pl.ds(r0 + g * _NB, _NB)], valsb)
            for b in range(_NB):
                pltpu.sync_copy(valsb.at[b], shared.at[idxv.at[g * _NB + b]],
                                add=True)
            return carry
        lax.fori_loop(0, _NG, grp, 0)

        plsc.subcore_barrier()

        @pl.when(s == 0)
        def _():
            pltpu.sync_copy(shared, out_r.at[c])

    return pl.kernel(body, out_type=out_type, mesh=_sc_mesh(),
                     scratch_types=scratch,
                     compiler_params=pltpu.CompilerParams(
                         use_tc_tiling_on_sc=False))


_BE = 8192
_BN = 3200
_BC = 8192


def _dot(a, b):
    return jnp.dot(a, b, preferred_element_type=_f32)


_PK = 8
_PROWS = N_EDGES // _PK
_BPR = _BE // _PK


def _full(shape):
    nd = len(shape)
    return pl.BlockSpec(shape, lambda i, _n=nd: (0,) * _n)


def _edge1_body(xg, eap, srcp, wu, lo_r, hi_r, rep,
                wxd, we, b1, w2, b2, o):
    v = xg[...]
    xsd = jnp.concatenate([v[0], v[1]], axis=1)
    srcb = _dot(srcp[...], rep[...])
    oh = ((srcb >= lo_r[...]) & (srcb < hi_r[...])).astype(_f32)
    h = (_dot(xsd, wxd[...])
         + _dot(eap[...], we[...]) + _dot(oh, wu[...]) + b1[...])
    h = jnp.maximum(h, 0.0)
    o[...] = _dot(h, w2[...]) + b2[...]


def _edge23_body(xg, v1, srcp, wu, lo_r, hi_r, rep,
                 wxd, we, b1, w2, b2, o):
    v = xg[...]
    xsd = jnp.concatenate([v[0], v[1]], axis=1)
    srcb = _dot(srcp[...], rep[...])
    oh = ((srcb >= lo_r[...]) & (srcb < hi_r[...])).astype(_f32)
    e1 = jnp.maximum(v1[...], 0.0)
    h = (_dot(xsd, wxd[...])
         + _dot(e1, we[...]) + _dot(oh, wu[...]) + b1[...])
    h = jnp.maximum(h, 0.0)
    o[...] = _dot(h, w2[...]) + b2[...]


def _edge_call(body, extra, extra_spec, weights, xgp, srcp, wu, lo_r, hi_r,
               rep):
    wspecs = [_full(w.shape) for w in weights]
    return pl.pallas_call(
        body,
        grid=(N_EDGES // _BE,),
        in_specs=([pl.BlockSpec((2, _BPR, 128), lambda i: (0, i, 0)),
                   extra_spec,
                   pl.BlockSpec((_BPR, _PK), lambda i: (i, 0)),
                   _full(wu.shape), _full((1, 32 * _PK)),
                   _full((1, 32 * _PK)), _full((_PK, 32 * _PK))] + wspecs),
        out_specs=pl.BlockSpec((_BPR, 128), lambda i: (i, 0)),
        out_shape=jax.ShapeDtypeStruct((_PROWS, 128), _f32),
    )(xgp, extra, srcp, wu, lo_r, hi_r, rep, *weights)


def _uprep_body(u, w1u, Vc, Hc, Mc, lo, hi, rg, wu_o, lo_o, hi_o):
    uproj = _dot(u[...], w1u[...])
    wu_o[...] = _dot(_dot(Vc[...], uproj), Hc[...]) * Mc[...]
    lo_o[...] = _dot(lo[...].astype(_f32), rg[...])
    hi_o[...] = _dot(hi[...].astype(_f32), rg[...])


def _uprep_call(u_in, w1u, Vc, Hc, Mc, lo, hi, rg):
    hid = w1u.shape[1]
    return pl.pallas_call(
        _uprep_body,
        grid=(1,),
        in_specs=[_full(u_in.shape), _full(w1u.shape), _full(Vc.shape),
                  _full(Hc.shape), _full(Mc.shape), _full((1, N_GRAPHS)),
                  _full((1, N_GRAPHS)), _full(rg.shape)],
        out_specs=[_full((32 * _PK, _PK * hid)), _full((1, 32 * _PK)),
                   _full((1, 32 * _PK))],
        out_shape=[jax.ShapeDtypeStruct((32 * _PK, _PK * hid), _f32),
                   jax.ShapeDtypeStruct((1, 32 * _PK), _f32),
                   jax.ShapeDtypeStruct((1, 32 * _PK), _f32)],
    )(u_in, w1u, Vc, Hc, Mc, lo, hi, rg)


_BB = 3200


def _bounds_body(bcol, tri, lo_o, hi_o, C):
    i = pl.program_id(0)
    oh = (bcol[...] == lax.broadcasted_iota(_i32, (_BB, N_GRAPHS), 1)
          ).astype(_f32)

    @pl.when(i == 0)
    def _():
        C[...] = jnp.zeros_like(C)

    C[...] += jnp.sum(oh, axis=0, keepdims=True)

    @pl.when(i == pl.num_programs(0) - 1)
    def _():
        cnt = C[...]
        lo = _dot(cnt, tri[...])
        lo_o[...] = lo.astype(_i32)
        hi_o[...] = (lo + cnt).astype(_i32)


def _bounds_call(bcol):
    tri = (lax.broadcasted_iota(_i32, (N_GRAPHS, N_GRAPHS), 0)
           < lax.broadcasted_iota(_i32, (N_GRAPHS, N_GRAPHS), 1)).astype(_f32)
    return pl.pallas_call(
        _bounds_body,
        grid=(N_NODES // _BB,),
        in_specs=[pl.BlockSpec((_BB, 1), lambda i: (i, 0)),
                  _full((N_GRAPHS, N_GRAPHS))],
        out_specs=[_full((1, N_GRAPHS)), _full((1, N_GRAPHS))],
        out_shape=[jax.ShapeDtypeStruct((1, N_GRAPHS), _i32),
                   jax.ShapeDtypeStruct((1, N_GRAPHS), _i32)],
        scratch_shapes=[pltpu.VMEM((1, N_GRAPHS), _f32)],
    )(bcol, tri)


def _node1_body(xr, aggr, bcol, bt8, u,
                w1x, w1a, w1u, b1, w2, b2, wgu, wgx, bg1, wg2, bg2,
                x1_o, u1_o, S, C):
    i = pl.program_id(0)
    a2 = aggr[...]
    aggs = a2[0] + a2[1]
    deg = jnp.maximum(aggs[:, 8:9], 1.0)
    aggn = aggs / deg
    oh = (bcol[...] == lax.broadcasted_iota(_i32, (_BN, N_GRAPHS), 1)
          ).astype(_f32)
    uproj = _dot(u[...], w1u[...])
    h = (_dot(xr[...], w1x[...]) + _dot(aggn, w1a[...]) + _dot(oh, uproj)
         + b1[...])
    h = jnp.maximum(h, 0.0)
    x2 = _dot(h, w2[...]) + b2[...]
    x1_o[...] = jnp.maximum(x2, 0.0)

    ohT = (lax.broadcasted_iota(_i32, (N_GRAPHS, _BN), 0) == bt8[0:1, :]
           ).astype(_f32)

    @pl.when(i == 0)
    def _():
        S[...] = jnp.zeros_like(S)
        C[...] = jnp.zeros_like(C)

    S[...] += _dot(ohT, x2)
    C[...] += jnp.sum(ohT, axis=1, keepdims=True)

    @pl.when(i == pl.num_programs(0) - 1)
    def _():
        Sm = S[...] / jnp.maximum(C[...], 1.0)
        hg = jnp.maximum(_dot(u[...], wgu[...]) + _dot(Sm, wgx[...])
                         + bg1[...], 0.0)
        u1_o[...] = jnp.maximum(_dot(hg, wg2[...]) + bg2[...], 0.0)


def _node23_body(xr, agg1r, agg2r, bcol, bt8, u,
                 nx2, na2, nu2, nb2, nw2, nc2,
                 nx3, na3, nu3, nb3, nw3, nc3,
                 gu2, gx2, gb2, gw2, gc2,
                 gu3, gx3, gb3, gw3, gc3,
                 mu_o, lv_o, S2, S3, C):
    i = pl.program_id(0)
    a1 = agg1r[...]
    deg = jnp.maximum((a1[0] + a1[1])[:, 8:9], 1.0)
    a2 = agg2r[...]
    a = (a2[0] + a2[1]) / deg
    oh = (bcol[...] == lax.broadcasted_iota(_i32, (_BN, N_GRAPHS), 1)
          ).astype(_f32)
    ohT = (lax.broadcasted_iota(_i32, (N_GRAPHS, _BN), 0) == bt8[0:1, :]
           ).astype(_f32)
    xv = xr[...]
    uproj2 = _dot(u[...], nu2[...])
    h2 = jnp.maximum(_dot(xv, nx2[...]) + _dot(a, na2[...]) + _dot(oh, uproj2)
                     + nb2[...], 0.0)
    x2b = _dot(h2, nw2[...]) + nc2[...]
    uproj3 = _dot(u[...], nu3[...])
    h3 = jnp.maximum(_dot(xv, nx3[...]) + _dot(a, na3[...]) + _dot(oh, uproj3)
                     + nb3[...], 0.0)
    x3b = _dot(h3, nw3[...]) + nc3[...]

    @pl.when(i == 0)
    def _():
        S2[...] = jnp.zeros_like(S2)
        S3[...] = jnp.zeros_like(S3)
        C[...] = jnp.zeros_like(C)

    S2[...] += _dot(ohT, x2b)
    S3[...] += _dot(ohT, x3b)
    C[...] += jnp.sum(ohT, axis=1, keepdims=True)

    @pl.when(i == pl.num_programs(0) - 1)
    def _():
        cc = jnp.maximum(C[...], 1.0)
        Sm2 = S2[...] / cc
        hg2 = jnp.maximum(_dot(u[...], gu2[...]) + _dot(Sm2, gx2[...])
                          + gb2[...], 0.0)
        mu_o[...] = _dot(hg2, gw2[...]) + gc2[...]
        Sm3 = S3[...] / cc
        hg3 = jnp.maximum(_dot(u[...], gu3[...]) + _dot(Sm3, gx3[...])
                          + gb3[...], 0.0)
        lv_o[...] = _dot(hg3, gw3[...]) + gc3[...]


def _dec_body(mu, lv, ep, w1, b1, w2, b2, o):
    std = jnp.exp(0.5 * lv[...])
    z = mu[...] + ep[...] * std
    h = jnp.maximum(_dot(z, w1[...]) + b1[...], 0.0)
    o[...] = jax.nn.sigmoid(_dot(h, w2[...]) + b2[...])


def _split_edge_w(W1):
    return W1[0:NF], W1[NF:2 * NF], W1[2 * NF:2 * NF + EF], W1[2 * NF + EF:]


def kernel(x, edge_index, edge_attr, u, batch, eps, params):
    src = edge_index[0].reshape(_NW, _RPW, _IW)
    dst = edge_index[1].reshape(_NW, _RPW, _IW)
    zeros16 = jnp.zeros((N_NODES, 16), _f32)
    bcol = batch.reshape(N_NODES, 1)
    bt8 = jnp.broadcast_to(batch.reshape(1, N_NODES), (8, N_NODES))

    eyeP = jnp.eye(_PK, dtype=_f32)

    def bd(w):
        return jnp.kron(eyeP, w)

    def tile_row(b):
        return jnp.tile(b.reshape(1, -1), (1, _PK))

    (e1W1, e1b1), (e1W2, e1b2) = params['gc1']['edge']
    w1x1, w1d1, w1e1, w1u1 = _split_edge_w(e1W1)
    w2p1 = jnp.concatenate([e1W2, jnp.zeros((HID, 8), _f32)], axis=1)
    b2p1 = jnp.concatenate([e1b2, jnp.ones((1,), _f32),
                            jnp.zeros((7,), _f32)])
    ew1 = [jnp.concatenate([bd(w1x1), bd(w1d1)], axis=0), bd(w1e1),
           tile_row(e1b1), bd(w2p1), tile_row(b2p1)]

    (e2W1, e2b1), (e2W2, e2b2) = params['gc2']['edge']
    (e3W1, e3b1), (e3W2, e3b2) = params['gc3']['edge']
    x2s, d2s, ee2, uu2 = _split_edge_w(e2W1)
    x3s, d3s, ee3, uu3 = _split_edge_w(e3W1)
    z64 = jnp.zeros((HID, 8), _f32)
    w1x23 = jnp.concatenate([x2s, x3s], axis=1)
    w1d23 = jnp.concatenate([d2s, d3s], axis=1)
    w1e23 = jnp.concatenate([jnp.concatenate([ee2, ee3], axis=1),
                             jnp.zeros((8, 2 * HID), _f32)], axis=0)
    w1u23 = jnp.concatenate([uu2, uu3], axis=1)
    b123 = jnp.concatenate([e2b1, e3b1])
    w2bd23 = jnp.concatenate([jnp.concatenate([e2W2, z64], axis=1),
                              jnp.concatenate([z64, e3W2], axis=1)], axis=0)
    b223 = jnp.concatenate([e2b2, e3b2])
    ew23 = [jnp.concatenate([bd(w1x23), bd(w1d23)], axis=0), bd(w1e23),
            tile_row(b123), bd(w2bd23), tile_row(b223)]

    Vc = jnp.kron(jnp.eye(N_GRAPHS, dtype=_f32), jnp.ones((_PK, 1), _f32))
    H1c = jnp.kron(jnp.ones((1, _PK), _f32), jnp.eye(HID, dtype=_f32))
    M1c = jnp.kron(jnp.ones((N_GRAPHS, 1), _f32),
                   jnp.kron(eyeP, jnp.ones((1, HID), _f32)))
    H23c = jnp.kron(jnp.ones((1, _PK), _f32),
                    jnp.eye(2 * HID, dtype=_f32))
    M23c = jnp.kron(jnp.ones((N_GRAPHS, 1), _f32),
                    jnp.kron(eyeP, jnp.ones((1, 2 * HID), _f32)))
    RGc = jnp.kron(jnp.eye(N_GRAPHS, dtype=_f32), jnp.ones((1, _PK), _f32))
    REPc = jnp.kron(jnp.ones((1, N_GRAPHS), _f32), eyeP)

    def node_w(p):
        (W1, b1), (W2, b2) = p
        return (W1[0:NF], W1[NF:NF + EF], W1[NF + EF:],
                b1.reshape(1, HID), W2, b2.reshape(1, NF))

    def glob_w(p):
        (W1, b1), (W2, b2) = p
        return (W1[0:UF], W1[UF:], b1.reshape(1, HID), W2,
                b2.reshape(1, UF))

    z8 = jnp.zeros((8, HID), _f32)
    nx1, na1, nu1, nb1, nw1, nc1 = node_w(params['gc1']['node'])
    na1p = jnp.concatenate([na1, z8], axis=0)
    g1 = glob_w(params['gc1']['global'])

    nx2, na2, nu2, nb2, nw2, nc2 = node_w(params['gc2']['node'])
    na2p = jnp.concatenate([na2, z8], axis=0)
    nx3, na3, nu3, nb3, nw3, nc3 = node_w(params['gc3']['node'])
    na3p = jnp.concatenate([z8, na3], axis=0)
    g2 = glob_w(params['gc2']['global'])
    g3 = glob_w(params['gc3']['global'])

    fc1W, fc1b = params['fc1']
    fc2W, fc2b = params['fc2']

    lo, hi = _bounds_call(bcol)
    wu1, lo_r, hi_r = _uprep_call(u, w1u1, Vc, H1c, M1c, lo, hi, RGc)
    srcp = edge_index[0].astype(_f32).reshape(_PROWS, _PK)
    eap = edge_attr.reshape(_PROWS, EF * _PK)
    gather = _make_gather()
    scatter = _make_scatter()
    xg1 = gather(x, src, dst).reshape(2, _PROWS, 128)
    ea_spec = pl.BlockSpec((_BPR, EF * _PK), lambda i: (i, 0))
    vals1 = _edge_call(_edge1_body, eap, ea_spec, ew1, xg1, srcp,
                       wu1, lo_r, hi_r, REPc)
    agg1 = scatter(vals1.reshape(_EROWS, _IW, 16), dst, zeros16)

    nspecs = [pl.BlockSpec((_BN, NF), lambda i: (i, 0)),
              pl.BlockSpec((2, _BN, 16), lambda i: (0, i, 0)),
              pl.BlockSpec((_BN, 1), lambda i: (i, 0)),
              pl.BlockSpec((8, _BN), lambda i: (0, i))]
    w_b1 = [nx1, na1p, nu1, nb1, nw1, nc1] + list(g1)
    x1, u1 = pl.pallas_call(
        _node1_body,
        grid=(N_NODES // _BN,),
        in_specs=nspecs + [_full(u.shape)] + [_full(w.shape) for w in w_b1],
        out_specs=[pl.BlockSpec((_BN, NF), lambda i: (i, 0)),
                   _full((N_GRAPHS, UF))],
        out_shape=[jax.ShapeDtypeStruct((N_NODES, NF), _f32),
                   jax.ShapeDtypeStruct((N_GRAPHS, UF), _f32)],
        scratch_shapes=[pltpu.VMEM((N_GRAPHS, NF), _f32),
                        pltpu.VMEM((N_GRAPHS, 1), _f32)],
    )(x, agg1, bcol, bt8, u, *w_b1)

    wu23, _, _ = _uprep_call(u1, w1u23, Vc, H23c, M23c, lo, hi, RGc)
    xg2 = gather(x1, src, dst).reshape(2, _PROWS, 128)
    v1_spec = pl.BlockSpec((_BPR, 128), lambda i: (i, 0))
    vals2 = _edge_call(_edge23_body, vals1, v1_spec, ew23, xg2, srcp,
                       wu23, lo_r, hi_r, REPc)
    agg2 = scatter(vals2.reshape(_EROWS, _IW, 16), dst, zeros16)

    w_b2 = [nx2, na2p, nu2, nb2, nw2, nc2,
            nx3, na3p, nu3, nb3, nw3, nc3] + list(g2) + list(g3)
    n2specs = [pl.BlockSpec((_BN, NF), lambda i: (i, 0)),
               pl.BlockSpec((2, _BN, 16), lambda i: (0, i, 0)),
               pl.BlockSpec((2, _BN, 16), lambda i: (0, i, 0)),
               pl.BlockSpec((_BN, 1), lambda i: (i, 0)),
               pl.BlockSpec((8, _BN), lambda i: (0, i))]
    mu, logvar = pl.pallas_call(
        _node23_body,
        grid=(N_NODES // _BN,),
        in_specs=n2specs + [_full(u.shape)] + [_full(w.shape) for w in w_b2],
        out_specs=[_full((N_GRAPHS, UF)), _full((N_GRAPHS, UF))],
        out_shape=[jax.ShapeDtypeStruct((N_GRAPHS, UF), _f32),
                   jax.ShapeDtypeStruct((N_GRAPHS, UF), _f32)],
        scratch_shapes=[pltpu.VMEM((N_GRAPHS, NF), _f32),
                        pltpu.VMEM((N_GRAPHS, NF), _f32),
                        pltpu.VMEM((N_GRAPHS, 1), _f32)],
    )(x1, agg1, agg2, bcol, bt8, u1, *w_b2)

    out = pl.pallas_call(
        _dec_body,
        grid=(pl.cdiv(OUT_COLS, _BC),),
        in_specs=[_full((N_GRAPHS, UF)), _full((N_GRAPHS, UF)),
                  _full((N_GRAPHS, UF)), _full(fc1W.shape),
                  _full((1, 100)),
                  pl.BlockSpec((100, _BC), lambda i: (0, i)),
                  pl.BlockSpec((1, _BC), lambda i: (0, i))],
        out_specs=pl.BlockSpec((N_GRAPHS, _BC), lambda i: (0, i)),
        out_shape=jax.ShapeDtypeStruct((N_GRAPHS, OUT_COLS), _f32),
    )(mu, logvar, eps, fc1W, fc1b.reshape(1, 100), fc2W,
      fc2b.reshape(1, OUT_COLS))

    return out, mu, logvar
